# baseline (speedup 1.0000x reference)
import jax
import jax.numpy as jnp
from jax import lax
from jax.experimental import pallas as pl
from jax.experimental.pallas import tpu as pltpu
from jax.experimental.pallas import tpu_sc as plsc

WD = 1e-4
B = 16384
D = 64
L = 16
NC = 2
NS = 16
NW = NC * NS
BPW = B // NW
C = 256
NCH = BPW // C


def _sc_body(emb_u, emb_i, users, pos, neg, raw_u, raw_i,
             scores_out, sq_out,
             idx_u, idx_p, idx_n, bu, bp, bn, bru, brp, brn,
             sc_buf, sq_buf, sem):
    cid = lax.axis_index("c")
    sid = lax.axis_index("s")
    wid = sid * NC + cid
    base = wid * BPW

    sq_acc = jnp.zeros((L,), jnp.float32)
    for c in range(NCH):
        row0 = base + c * C
        pltpu.sync_copy(users.at[pl.ds(row0, C)], idx_u)
        pltpu.sync_copy(pos.at[pl.ds(row0, C)], idx_p)
        pltpu.sync_copy(neg.at[pl.ds(row0, C)], idx_n)
        cps = [
            pltpu.async_copy(emb_u.at[idx_u], bu, sem),
            pltpu.async_copy(emb_i.at[idx_p], bp, sem),
            pltpu.async_copy(emb_i.at[idx_n], bn, sem),
            pltpu.async_copy(raw_u.at[idx_u], bru, sem),
            pltpu.async_copy(raw_i.at[idx_p], brp, sem),
            pltpu.async_copy(raw_i.at[idx_n], brn, sem),
        ]
        for cp in cps:
            cp.wait()

        def row(i, sqa):
            acc = jnp.zeros((L,), jnp.float32)
            for k in range(D // L):
                s = pl.ds(k * L, L)
                uu = bu[i, s]
                dd = bn[i, s] - bp[i, s]
                acc = acc + uu * dd
                ru = bru[i, s]
                rp = brp[i, s]
                rn = brn[i, s]
                sqa = sqa + ru * ru + rp * rp + rn * rn
            sc_buf[i, :] = acc
            return sqa

        sq_acc = lax.fori_loop(0, C, row, sq_acc)
        pltpu.sync_copy(sc_buf, scores_out.at[pl.ds(row0, C)])

    sq_buf[...] = sq_acc
    pltpu.sync_copy(sq_buf, sq_out.at[wid])


def _tc_body(sc_ref, sq_ref, out_ref):
    x = jnp.sum(sc_ref[...], axis=1, keepdims=True)
    sp = jnp.maximum(x, 0.0) + jnp.log1p(jnp.exp(-jnp.abs(x)))
    reg = jnp.sum(sq_ref[...])
    out_ref[0, 0] = jnp.sum(sp) / B + (0.5 * WD / B) * reg


def kernel(emb_users, emb_items, users, pos_items, neg_items,
           raw_emb_users, raw_emb_items):
    users = users.astype(jnp.int32)
    pos_items = pos_items.astype(jnp.int32)
    neg_items = neg_items.astype(jnp.int32)

    mesh = plsc.VectorSubcoreMesh(
        core_axis_name="c", subcore_axis_name="s",
        num_cores=NC, num_subcores=NS)
    sc = pl.kernel(
        _sc_body,
        out_type=[
            jax.ShapeDtypeStruct((B, L), jnp.float32),
            jax.ShapeDtypeStruct((NW, L), jnp.float32),
        ],
        mesh=mesh,
        scratch_types=[
            pltpu.VMEM((C,), jnp.int32),
            pltpu.VMEM((C,), jnp.int32),
            pltpu.VMEM((C,), jnp.int32),
            pltpu.VMEM((C, D), jnp.float32),
            pltpu.VMEM((C, D), jnp.float32),
            pltpu.VMEM((C, D), jnp.float32),
            pltpu.VMEM((C, D), jnp.float32),
            pltpu.VMEM((C, D), jnp.float32),
            pltpu.VMEM((C, D), jnp.float32),
            pltpu.VMEM((C, L), jnp.float32),
            pltpu.VMEM((L,), jnp.float32),
            pltpu.SemaphoreType.DMA,
        ],
        compiler_params=pltpu.CompilerParams(use_tc_tiling_on_sc=False),
    )
    scores_mat, sq = sc(emb_users, emb_items, users, pos_items, neg_items,
                        raw_emb_users, raw_emb_items)

    out = pl.pallas_call(
        _tc_body,
        out_shape=jax.ShapeDtypeStruct((1, 1), jnp.float32),
        out_specs=pl.BlockSpec(memory_space=pltpu.SMEM),
    )(scores_mat, sq)
    return out[0, 0]


# speedup vs baseline: 2.0129x; 2.0129x over previous
import jax
import jax.numpy as jnp
from jax import lax
from jax.experimental import pallas as pl
from jax.experimental.pallas import tpu as pltpu
from jax.experimental.pallas import tpu_sc as plsc

WD = 1e-4
B = 16384
D = 64
L = 16
NC = 2
NS = 16
NW = NC * NS
BPW = B // NW
G = 16
NG = BPW // G
NBLK = 125000


def _sc_body(emb_u3, emb_i3, users, pos, neg, raw_u3, raw_i3,
             scores_out, sq_out,
             idxu, idxp, idxn,
             bu, bp, bn, bru, brp, brn, scb, sqb, sem):
    cid = lax.axis_index("c")
    sid = lax.axis_index("s")
    wid = sid * NC + cid
    base = wid * BPW

    pltpu.sync_copy(users.at[pl.ds(base, BPW)], idxu)
    pltpu.sync_copy(pos.at[pl.ds(base, BPW)], idxp)
    pltpu.sync_copy(neg.at[pl.ds(base, BPW)], idxn)

    lane = lax.iota(jnp.int32, L)

    def group(c, sq_acc):
        s16 = pl.ds(c * G, G)
        ivu = idxu[s16]
        ivp = idxp[s16]
        ivn = idxn[s16]
        bkvu = lax.shift_right_logical(ivu, 3)
        bkvp = lax.shift_right_logical(ivp, 3)
        bkvn = lax.shift_right_logical(ivn, 3)
        cps = []
        for j in range(G):
            cps.append(pltpu.async_copy(emb_u3.at[bkvu[j]], bu.at[j], sem))
            cps.append(pltpu.async_copy(emb_i3.at[bkvp[j]], bp.at[j], sem))
            cps.append(pltpu.async_copy(emb_i3.at[bkvn[j]], bn.at[j], sem))
            cps.append(pltpu.async_copy(raw_u3.at[bkvu[j]], bru.at[j], sem))
            cps.append(pltpu.async_copy(raw_i3.at[bkvp[j]], brp.at[j], sem))
            cps.append(pltpu.async_copy(raw_i3.at[bkvn[j]], brn.at[j], sem))
        for cp in cps:
            cp.wait()

        sqv = sq_acc
        for j in range(G):
            ru = jnp.bitwise_and(ivu[j], 7)
            rp = jnp.bitwise_and(ivp[j], 7)
            rn = jnp.bitwise_and(ivn[j], 7)
            acc = jnp.zeros((L,), jnp.float32)
            for k in range(D // L):
                s = pl.ds(k * L, L)
                uv = bu[j, ru, s]
                pv = bp[j, rp, s]
                nv = bn[j, rn, s]
                acc = acc + uv * (nv - pv)
                av = bru[j, ru, s]
                bv = brp[j, rp, s]
                cv = brn[j, rn, s]
                sqv = sqv + av * av + bv * bv + cv * cv
            scb[j, pl.ds(0, L)] = acc
        pltpu.sync_copy(scb, scores_out.at[pl.ds(base + c * G, G)])
        return sqv

    sq_acc = lax.fori_loop(0, NG, group, jnp.zeros((L,), jnp.float32))
    sqb[...] = sq_acc
    pltpu.sync_copy(sqb, sq_out.at[pl.ds(wid * L, L)])


def _tc_body(sc_ref, sq_ref, out_ref):
    x = jnp.sum(sc_ref[:, 0:L], axis=1, keepdims=True)
    sp = jnp.maximum(x, 0.0) + jnp.log1p(jnp.exp(-jnp.abs(x)))
    reg = jnp.sum(sq_ref[...])
    out_ref[0, 0] = jnp.sum(sp) / B + (0.5 * WD / B) * reg


def kernel(emb_users, emb_items, users, pos_items, neg_items,
           raw_emb_users, raw_emb_items):
    users = users.astype(jnp.int32)
    pos_items = pos_items.astype(jnp.int32)
    neg_items = neg_items.astype(jnp.int32)
    emb_u3 = emb_users.reshape(NBLK, 8, D)
    emb_i3 = emb_items.reshape(NBLK, 8, D)
    raw_u3 = raw_emb_users.reshape(NBLK, 8, D)
    raw_i3 = raw_emb_items.reshape(NBLK, 8, D)

    mesh = plsc.VectorSubcoreMesh(
        core_axis_name="c", subcore_axis_name="s",
        num_cores=NC, num_subcores=NS)
    sc = pl.kernel(
        _sc_body,
        out_type=[
            jax.ShapeDtypeStruct((B, 128), jnp.float32),
            jax.ShapeDtypeStruct((NW * L,), jnp.float32),
        ],
        mesh=mesh,
        scratch_types=[
            pltpu.VMEM((BPW,), jnp.int32),
            pltpu.VMEM((BPW,), jnp.int32),
            pltpu.VMEM((BPW,), jnp.int32),
            pltpu.VMEM((G, 8, D), jnp.float32),
            pltpu.VMEM((G, 8, D), jnp.float32),
            pltpu.VMEM((G, 8, D), jnp.float32),
            pltpu.VMEM((G, 8, D), jnp.float32),
            pltpu.VMEM((G, 8, D), jnp.float32),
            pltpu.VMEM((G, 8, D), jnp.float32),
            pltpu.VMEM((G, 128), jnp.float32),
            pltpu.VMEM((L,), jnp.float32),
            pltpu.SemaphoreType.DMA,
        ],
    )
    scores, sq = sc(emb_u3, emb_i3, users, pos_items, neg_items,
                    raw_u3, raw_i3)

    out = pl.pallas_call(
        _tc_body,
        out_shape=jax.ShapeDtypeStruct((1, 1), jnp.float32),
        out_specs=pl.BlockSpec(memory_space=pltpu.SMEM),
    )(scores, sq.reshape(4, 128))
    return out[0, 0]
